# baseline (speedup 1.0000x reference)
import functools

import jax
import jax.numpy as jnp
import ml_dtypes
import numpy as np
from jax import lax
from jax.experimental import pallas as pl
from jax.experimental.pallas import tpu as pltpu
from jax.experimental.pallas import tpu_sc as plsc

NC = 2
NS = 16
NW = NC * NS
LANES = 16

POS_SCALE = 1.0


def _pos_table(num_positions, m):
    pos = np.arange(num_positions, dtype=np.float64)
    denom = 10000.0 ** np.linspace(0.0, 1.0, m)
    arg = pos[:, None] / denom[None, :]
    tbl = np.zeros((num_positions, m), dtype=np.float32)
    tbl[:, ::2] = np.sin(arg[:, ::2])
    tbl[:, 1::2] = np.cos(arg[:, 1::2])
    return tbl


def _make_sc_embed(B, C, M):
    cols_per_w = C // NW
    CHUNK = 16
    n_groups = cols_per_w // CHUNK
    n_chunks = n_groups * B
    N_BUF = 6
    GLOOK = 2

    mesh = plsc.VectorSubcoreMesh(
        core_axis_name="c", subcore_axis_name="s",
        num_cores=NC, num_subcores=NS)

    @functools.partial(
        pl.kernel,
        mesh=mesh,
        out_type=jax.ShapeDtypeStruct((B, C, M), jnp.float32),
        scratch_types=[
            pltpu.VMEM((B * cols_per_w,), jnp.int32),
            pltpu.VMEM((1, CHUNK, M), jnp.float32),
            pltpu.VMEM((N_BUF, CHUNK, M), jnp.float32),
            pltpu.SemaphoreType.DMA((1,)),
            pltpu.SemaphoreType.DMA((N_BUF,)),
            pltpu.SemaphoreType.DMA((N_BUF,)),
        ],
    )
    def body(inp_hbm, emb_hbm, pos_hbm, out_hbm,
             idx_v, pos_v, gath_v, psem, gsem, ssem):
        wid = lax.axis_index("s") * NC + lax.axis_index("c")
        c0 = wid * cols_per_w

        for b in range(B):
            pltpu.sync_copy(inp_hbm.at[b, pl.ds(c0, cols_per_w)],
                            idx_v.at[pl.ds(b * cols_per_w, cols_per_w)])

        pos_d = [None] * n_groups
        gath_d = [None] * n_chunks
        store_d = [None] * n_chunks

        def issue_pos(h):
            pos_d[h] = pltpu.async_copy(
                pos_hbm.at[pl.ds(c0 + h * CHUNK, CHUNK)],
                pos_v.at[0], psem.at[0])

        def issue_gather(k):
            h, b = divmod(k, B)
            off = b * cols_per_w + h * CHUNK
            gath_d[k] = pltpu.async_copy(
                emb_hbm.at[idx_v.at[pl.ds(off, CHUNK)]],
                gath_v.at[k % N_BUF], gsem.at[k % N_BUF])

        def issue_store(k):
            h, b = divmod(k, B)
            store_d[k] = pltpu.async_copy(
                gath_v.at[k % N_BUF],
                out_hbm.at[b, pl.ds(c0 + h * CHUNK, CHUNK)],
                ssem.at[k % N_BUF])

        issue_pos(0)
        for k in range(GLOOK):
            issue_gather(k)

        for k in range(n_chunks):
            j = k + GLOOK
            if j < n_chunks:
                if j >= N_BUF:
                    store_d[j - N_BUF].wait()
                issue_gather(j)
            h, b = divmod(k, B)
            if b == 0:
                pos_d[h].wait()
            gath_d[k].wait()

            pbuf = 0

            def add_body(r, _):
                for jj in range(M // LANES):
                    plsc.addupdate(
                        gath_v.at[k % N_BUF, r, pl.ds(jj * LANES, LANES)],
                        pos_v[pbuf, r, pl.ds(jj * LANES, LANES)])
                return _
            lax.fori_loop(0, CHUNK, add_body, None)

            issue_store(k)
            if b == B - 1 and h + 1 < n_groups:
                issue_pos(h + 1)

        for k in range(n_chunks - N_BUF, n_chunks):
            if store_d[k] is not None and k >= 0:
                store_d[k].wait()

    return body


def kernel(input, emb):
    B, C = input.shape
    M = emb.shape[1]
    pos_bf = jnp.asarray(
        (_pos_table(C, M) * POS_SCALE).astype(ml_dtypes.bfloat16))
    pos = lax.optimization_barrier(pos_bf).astype(jnp.float32)
    return _make_sc_embed(B, C, M)(input.astype(jnp.int32), emb, pos)


# speedup vs baseline: 1.0665x; 1.0665x over previous
import functools

import jax
import jax.numpy as jnp
import ml_dtypes
import numpy as np
from jax import lax
from jax.experimental import pallas as pl
from jax.experimental.pallas import tpu as pltpu
from jax.experimental.pallas import tpu_sc as plsc

NC = 2
NS = 16
NW = NC * NS
LANES = 16

POS_SCALE = 1.0


def _pos_table(num_positions, m):
    pos = np.arange(num_positions, dtype=np.float64)
    denom = 10000.0 ** np.linspace(0.0, 1.0, m)
    arg = pos[:, None] / denom[None, :]
    tbl = np.zeros((num_positions, m), dtype=np.float32)
    tbl[:, ::2] = np.sin(arg[:, ::2])
    tbl[:, 1::2] = np.cos(arg[:, 1::2])
    return tbl


def _make_sc_embed(B, C, M):
    cols_per_w = C // NW
    PCH = 16
    BP = 2
    n_groups = cols_per_w // PCH
    n_pairs = B // BP
    n_chunks = n_groups * n_pairs
    N_BUF = 3
    GLOOK = 2

    mesh = plsc.VectorSubcoreMesh(
        core_axis_name="c", subcore_axis_name="s",
        num_cores=NC, num_subcores=NS)

    @functools.partial(
        pl.kernel,
        mesh=mesh,
        out_type=jax.ShapeDtypeStruct((B, C, M), jnp.float32),
        scratch_types=[
            pltpu.VMEM((B * cols_per_w,), jnp.int32),
            pltpu.VMEM((1, PCH, M), jnp.float32),
            pltpu.VMEM((N_BUF, BP * PCH, M), jnp.float32),
            pltpu.SemaphoreType.DMA((1,)),
            pltpu.SemaphoreType.DMA((N_BUF,)),
            pltpu.SemaphoreType.DMA((N_BUF,)),
        ],
    )
    def body(inp_hbm, emb_hbm, pos_hbm, out_hbm,
             idx_v, pos_v, gath_v, psem, gsem, ssem):
        wid = lax.axis_index("s") * NC + lax.axis_index("c")
        c0 = wid * cols_per_w

        for h in range(n_groups):
            for b in range(B):
                pltpu.sync_copy(
                    inp_hbm.at[b, pl.ds(c0 + h * PCH, PCH)],
                    idx_v.at[pl.ds(h * B * PCH + b * PCH, PCH)])

        pos_d = [None] * n_groups
        gath_d = [None] * n_chunks
        store_d = [[None, None] for _ in range(n_chunks)]

        def issue_pos(h):
            pos_d[h] = pltpu.async_copy(
                pos_hbm.at[pl.ds(c0 + h * PCH, PCH)],
                pos_v.at[0], psem.at[0])

        def issue_gather(k):
            h, bp = divmod(k, n_pairs)
            off = h * B * PCH + bp * BP * PCH
            gath_d[k] = pltpu.async_copy(
                emb_hbm.at[idx_v.at[pl.ds(off, BP * PCH)]],
                gath_v.at[k % N_BUF], gsem.at[k % N_BUF])

        def issue_store(k):
            h, bp = divmod(k, n_pairs)
            for d in range(BP):
                store_d[k][d] = pltpu.async_copy(
                    gath_v.at[k % N_BUF, pl.ds(d * PCH, PCH)],
                    out_hbm.at[bp * BP + d, pl.ds(c0 + h * PCH, PCH)],
                    ssem.at[k % N_BUF])

        issue_pos(0)
        for k in range(GLOOK):
            issue_gather(k)

        for k in range(n_chunks):
            j = k + GLOOK
            if j < n_chunks:
                if j >= N_BUF:
                    for d in store_d[j - N_BUF]:
                        d.wait()
                issue_gather(j)
            h, bp = divmod(k, n_pairs)
            if bp == 0:
                pos_d[h].wait()
            gath_d[k].wait()

            pbuf = 0

            def add_body(r, _):
                for jj in range(M // LANES):
                    pvec = pos_v[pbuf, r, pl.ds(jj * LANES, LANES)]
                    plsc.addupdate(
                        gath_v.at[k % N_BUF, r, pl.ds(jj * LANES, LANES)],
                        pvec)
                    plsc.addupdate(
                        gath_v.at[k % N_BUF, PCH + r,
                                  pl.ds(jj * LANES, LANES)], pvec)
                return _
            lax.fori_loop(0, PCH, add_body, None)

            issue_store(k)
            if bp == n_pairs - 1 and h + 1 < n_groups:
                issue_pos(h + 1)

        for k in range(max(0, n_chunks - N_BUF), n_chunks):
            for d in store_d[k]:
                if d is not None:
                    d.wait()

    return body


def kernel(input, emb):
    B, C = input.shape
    M = emb.shape[1]
    pos_bf = jnp.asarray(
        (_pos_table(C, M) * POS_SCALE).astype(ml_dtypes.bfloat16))
    pos = lax.optimization_barrier(pos_bf).astype(jnp.float32)
    return _make_sc_embed(B, C, M)(input.astype(jnp.int32), emb, pos)


# speedup vs baseline: 1.0712x; 1.0044x over previous
import functools

import jax
import jax.numpy as jnp
import ml_dtypes
import numpy as np
from jax import lax
from jax.experimental import pallas as pl
from jax.experimental.pallas import tpu as pltpu
from jax.experimental.pallas import tpu_sc as plsc

NC = 2
NS = 16
NW = NC * NS
LANES = 16

POS_SCALE = 1.0


def _pos_table(num_positions, m):
    pos = np.arange(num_positions, dtype=np.float64)
    denom = 10000.0 ** np.linspace(0.0, 1.0, m)
    arg = pos[:, None] / denom[None, :]
    tbl = np.zeros((num_positions, m), dtype=np.float32)
    tbl[:, ::2] = np.sin(arg[:, ::2])
    tbl[:, 1::2] = np.cos(arg[:, 1::2])
    return tbl


def _make_sc_embed(B, C, M):
    cols_per_w = C // NW
    CHUNK = 16
    n_groups = cols_per_w // CHUNK
    n_chunks = n_groups * B
    N_BUF = 5
    GLOOK = 2

    mesh = plsc.VectorSubcoreMesh(
        core_axis_name="c", subcore_axis_name="s",
        num_cores=NC, num_subcores=NS)

    @functools.partial(
        pl.kernel,
        mesh=mesh,
        out_type=jax.ShapeDtypeStruct((B, C, M), jnp.float32),
        scratch_types=[
            pltpu.VMEM((B * cols_per_w,), jnp.int32),
            pltpu.VMEM((2, CHUNK, M), jnp.float32),
            pltpu.VMEM((N_BUF, CHUNK, M), jnp.float32),
            pltpu.SemaphoreType.DMA((2,)),
            pltpu.SemaphoreType.DMA((N_BUF,)),
            pltpu.SemaphoreType.DMA((N_BUF,)),
        ],
    )
    def body(inp_hbm, emb_hbm, pos_hbm, out_hbm,
             idx_v, pos_v, gath_v, psem, gsem, ssem):
        wid = lax.axis_index("s") * NC + lax.axis_index("c")
        c0 = wid * cols_per_w

        for b in range(B):
            pltpu.sync_copy(inp_hbm.at[b, pl.ds(c0, cols_per_w)],
                            idx_v.at[pl.ds(b * cols_per_w, cols_per_w)])

        pos_d = [None] * n_groups
        gath_d = [None] * n_chunks
        store_d = [None] * n_chunks

        def issue_pos(h):
            pos_d[h] = pltpu.async_copy(
                pos_hbm.at[pl.ds(c0 + h * CHUNK, CHUNK)],
                pos_v.at[h % 2], psem.at[h % 2])

        def issue_gather(k):
            h, b = divmod(k, B)
            off = b * cols_per_w + h * CHUNK
            gath_d[k] = pltpu.async_copy(
                emb_hbm.at[idx_v.at[pl.ds(off, CHUNK)]],
                gath_v.at[k % N_BUF], gsem.at[k % N_BUF])

        def issue_store(k):
            h, b = divmod(k, B)
            store_d[k] = pltpu.async_copy(
                gath_v.at[k % N_BUF],
                out_hbm.at[b, pl.ds(c0 + h * CHUNK, CHUNK)],
                ssem.at[k % N_BUF])

        issue_pos(0)
        if n_groups > 1:
            issue_pos(1)
        for k in range(GLOOK):
            issue_gather(k)

        for k in range(n_chunks):
            j = k + GLOOK
            if j < n_chunks:
                if j >= N_BUF:
                    store_d[j - N_BUF].wait()
                issue_gather(j)
            h, b = divmod(k, B)
            if b == 0:
                pos_d[h].wait()
            gath_d[k].wait()

            pbuf = h % 2

            def add_body(r, _):
                for jj in range(M // LANES):
                    plsc.addupdate(
                        gath_v.at[k % N_BUF, r, pl.ds(jj * LANES, LANES)],
                        pos_v[pbuf, r, pl.ds(jj * LANES, LANES)])
                return _
            lax.fori_loop(0, CHUNK, add_body, None)

            issue_store(k)
            if b == B - 1 and h + 2 < n_groups:
                issue_pos(h + 2)

        for k in range(n_chunks - N_BUF, n_chunks):
            if store_d[k] is not None and k >= 0:
                store_d[k].wait()

    return body


def kernel(input, emb):
    B, C = input.shape
    M = emb.shape[1]
    pos_bf = jnp.asarray(
        (_pos_table(C, M) * POS_SCALE).astype(ml_dtypes.bfloat16))
    pos = lax.optimization_barrier(pos_bf).astype(jnp.float32)
    return _make_sc_embed(B, C, M)(input.astype(jnp.int32), emb, pos)
